# baseline (speedup 1.0000x reference)
import functools

import jax
import jax.numpy as jnp
from jax import lax
from jax.experimental import pallas as pl
from jax.experimental.pallas import tpu as pltpu
from jax.experimental.pallas import tpu_sc as plsc

N = 100000
E = 1600000
T = 12
H = 32
HORIZON = 12

NC = 2
NS = 16
HALF = N // 2

NBATCH = 6250
BE = 256
NCHUNK = 2
CH = 128

f32 = jnp.float32
i32 = jnp.int32


def _iota16():
    return lax.iota(i32, 16)


def _gru_body(x_ref, emb_ref, wenc_ref, benc_ref, wi_ref, wh_ref, bi_ref,
              bh_ref, out_ref):
    wi = wi_ref[...]
    wh = wh_ref[...]
    wv = jnp.dot(wenc_ref[...], wi, preferred_element_type=f32)
    c = jnp.dot(emb_ref[...] + benc_ref[...], wi,
                preferred_element_type=f32) + bi_ref[...]
    x = x_ref[...]
    nb = x.shape[0]
    h = jnp.zeros((nb, H), f32)
    for t in range(T):
        gi = x[:, t:t + 1] * wv + c
        gh = jnp.dot(h, wh, preferred_element_type=f32) + bh_ref[...]
        r = jax.nn.sigmoid(gi[:, :H] + gh[:, :H])
        z = jax.nn.sigmoid(gi[:, H:2 * H] + gh[:, H:2 * H])
        n = jnp.tanh(gi[:, 2 * H:] + r * gh[:, 2 * H:])
        h = (1.0 - z) * n + z * h
    out_ref[...] = h


def _run_gru(x2, node_emb, W_enc, b_enc, Wi, Wh, bi, bh):
    NB = 5000
    grid = N // NB
    return pl.pallas_call(
        _gru_body,
        grid=(grid,),
        in_specs=[
            pl.BlockSpec((NB, T), lambda i: (i, 0)),
            pl.BlockSpec((NB, H), lambda i: (i, 0)),
            pl.BlockSpec((1, H), lambda i: (0, 0)),
            pl.BlockSpec((1, H), lambda i: (0, 0)),
            pl.BlockSpec((H, 3 * H), lambda i: (0, 0)),
            pl.BlockSpec((H, 3 * H), lambda i: (0, 0)),
            pl.BlockSpec((1, 3 * H), lambda i: (0, 0)),
            pl.BlockSpec((1, 3 * H), lambda i: (0, 0)),
        ],
        out_specs=pl.BlockSpec((NB, H), lambda i: (i, 0)),
        out_shape=jax.ShapeDtypeStruct((N, H), f32),
    )(x2, node_emb, W_enc, b_enc, Wi, Wh, bi, bh)


DEG_W = 16
DEG_SUB = 6400
DEG_SUB_LAST = N - 15 * DEG_SUB
DEG_CHUNK = 200


def _deg_body(ed_ref, wrep_ref, inv_ref, acc_sh, e0, e1, e2, w0, w1, w2,
              zbuf, dbuf, se0, se1, se2, sw0, sw1, sw2, ss0, ss1):
    c = lax.axis_index("c")
    s = lax.axis_index("s")
    zeros16f = jnp.zeros((16,), f32)
    srow = 1 - c
    ebufs, esems = [e0, e1, e2], [se0, se1, se2]
    wbufs, wsems = [w0, w1, w2], [sw0, sw1, sw2]
    ssems = [ss0, ss1]

    def zb(i, _):
        zbuf[i, :] = zeros16f
        return 0
    lax.fori_loop(0, ZB, zb, 0)
    zbase = s * DEG_SUB
    zrows = jnp.where(s == 15, DEG_SUB_LAST, DEG_SUB)

    def zc(k, _):
        off = jnp.minimum(k * ZB, zrows - ZB)
        pltpu.sync_copy(zbuf, acc_sh.at[pl.ds(zbase + off, ZB)])
        return 0
    lax.fori_loop(0, DEG_SUB // ZB, zc, 0)
    plsc.subcore_barrier()

    nb = 781 + jnp.where(s < 4, 1, 0)
    start_s = s * 781 + jnp.minimum(s, 4)

    def fire_edge(b, ei):
        pltpu.async_copy(ed_ref.at[start_s + b], ebufs[ei], esems[ei])

    def wait_edge(ei):
        pltpu.make_async_copy(ed_ref.at[0], ebufs[ei], esems[ei]).wait()

    def fire_wrep(b, wi):
        pltpu.async_copy(wrep_ref.at[start_s + b], wbufs[wi], wsems[wi])

    def wait_wrep(wi):
        pltpu.make_async_copy(wrep_ref.at[0], wbufs[wi], wsems[wi]).wait()

    def fire_scatter(ei, ri):
        pltpu.async_copy(wbufs[ei], acc_sh.at[ebufs[ei].at[srow]],
                         ssems[ri], add=True)

    def wait_scatter(ei, ri):
        pltpu.make_async_copy(wbufs[ei], acc_sh.at[pl.ds(0, CH)],
                              ssems[ri]).wait()

    fire_edge(0, 0)
    fire_wrep(0, 0)
    fire_edge(1, 1)
    fire_wrep(1, 1)

    def stage_group(k, _):
        for i in range(6):
            b = k * 6 + i

            @pl.when((b >= 1) & (b <= nb))
            def _():
                wait_scatter((i + 2) % 3, (i + 1) % 2)

            @pl.when(b + 2 < nb)
            def _():
                fire_edge(b + 2, (i + 2) % 3)
                fire_wrep(b + 2, (i + 2) % 3)

            @pl.when(b < nb)
            def _():
                wait_edge(i % 3)
                wait_wrep(i % 3)
                fire_scatter(i % 3, i % 2)
        return 0
    lax.fori_loop(0, (nb + 1 + 5) // 6, stage_group, 0)
    plsc.subcore_barrier()

    base = s * DEG_SUB
    for k in range(DEG_SUB // DEG_CHUNK):
        off = base + jnp.minimum(k * DEG_CHUNK, zrows - DEG_CHUNK)
        pltpu.sync_copy(acc_sh.at[pl.ds(off, DEG_CHUNK)], dbuf)

        def ir(r, _):
            deg = dbuf[r, :]
            dbuf[r, :] = 1.0 / jnp.where(deg == 0.0, 1.0, deg)
            return 0
        lax.fori_loop(0, DEG_CHUNK, ir, 0)
        pltpu.sync_copy(dbuf, inv_ref.at[pl.ds(c * N + off, DEG_CHUNK)])


def _run_deg(ed, wrep):
    mesh = plsc.VectorSubcoreMesh(core_axis_name="c", subcore_axis_name="s")
    return pl.kernel(
        _deg_body,
        out_type=jax.ShapeDtypeStruct((2 * N, DEG_W), f32),
        mesh=mesh,
        scratch_types=[
            pltpu.VMEM_SHARED((N, DEG_W), f32),
            pltpu.VMEM((3, CH), i32),
            pltpu.VMEM((3, CH), i32),
            pltpu.VMEM((3, CH), i32),
            pltpu.VMEM((CH, DEG_W), f32),
            pltpu.VMEM((CH, DEG_W), f32),
            pltpu.VMEM((CH, DEG_W), f32),
            pltpu.VMEM((ZB, DEG_W), f32),
            pltpu.VMEM((DEG_CHUNK, DEG_W), f32),
            pltpu.SemaphoreType.DMA,
            pltpu.SemaphoreType.DMA,
            pltpu.SemaphoreType.DMA,
            pltpu.SemaphoreType.DMA,
            pltpu.SemaphoreType.DMA,
            pltpu.SemaphoreType.DMA,
            pltpu.SemaphoreType.DMA,
            pltpu.SemaphoreType.DMA,
        ],
        compiler_params=pltpu.CompilerParams(use_tc_tiling_on_sc=False,
                                             needs_layout_passes=False),
    )(ed, wrep)


ACC_ROWS = 50176
OUT_SUB = 3200
OUT_SUB_LAST = HALF - 15 * OUT_SUB
ZB = 16
NB2 = E // CH


def _hop_body(x_ref, ed_ref, inv_ref, out_ref, acc_sh,
              e0, e1, e2, r0, r1, zbuf, inv_v,
              se0, se1, se2, sg0, sg1, ss0, ss1):
    c = lax.axis_index("c")
    s = lax.axis_index("s")
    iota = _iota16()
    zeros16f = jnp.zeros((16,), f32)
    coff = c * HALF
    ebufs, esems = [e0, e1, e2], [se0, se1, se2]
    rbufs, gsems, ssems = [r0, r1], [sg0, sg1], [ss0, ss1]

    def zb(i, _):
        zbuf[i, pl.ds(0, 16)] = zeros16f
        zbuf[i, pl.ds(16, 16)] = zeros16f
        return 0
    lax.fori_loop(0, ZB, zb, 0)
    zrows = ACC_ROWS // NS

    def zc(k, _):
        pltpu.sync_copy(zbuf, acc_sh.at[pl.ds(s * zrows + k * ZB, ZB)])
        return 0
    lax.fori_loop(0, zrows // ZB, zc, 0)
    plsc.subcore_barrier()

    nb = 781 + jnp.where(s < 4, 1, 0)
    start_s = s * 781 + jnp.minimum(s, 4)

    def fire_edge(b, ei):
        pltpu.async_copy(ed_ref.at[start_s + b], ebufs[ei], esems[ei])

    def wait_edge(ei):
        pltpu.make_async_copy(ed_ref.at[0], ebufs[ei], esems[ei]).wait()

    def fire_gather(ei, ri):
        pltpu.async_copy(x_ref.at[ebufs[ei].at[0]], rbufs[ri], gsems[ri])

    def wait_gather(ri):
        pltpu.make_async_copy(x_ref.at[pl.ds(0, CH)], rbufs[ri],
                              gsems[ri]).wait()

    def fire_scatter(ei, ri):
        pltpu.async_copy(rbufs[ri], acc_sh.at[ebufs[ei].at[1]], ssems[ri],
                         add=True)

    def wait_scatter(ri):
        pltpu.make_async_copy(rbufs[ri], acc_sh.at[pl.ds(0, CH)],
                              ssems[ri]).wait()

    def compute(ei, ri):
        es, rs = ebufs[ei], rbufs[ri]

        def wm(g, _):
            sl = pl.ds(g * 16, 16)
            dst = es[1, sl]
            local = dst - coff
            ok = (local >= 0) & (local < HALF)
            trash = HALF + ((s * 16 + g * 16 + iota) & 127)
            es[1, sl] = jnp.where(ok, local, trash)
            w16 = plsc.bitcast(es[2, sl], f32)
            for i in range(16):
                wsp = jnp.full((16,), w16[i], f32)
                r = g * 16 + i
                rs[r, pl.ds(0, 16)] = rs[r, pl.ds(0, 16)] * wsp
                rs[r, pl.ds(16, 16)] = rs[r, pl.ds(16, 16)] * wsp
            return 0
        lax.fori_loop(0, 8, wm, 0)

    fire_edge(0, 0)
    fire_edge(1, 1)
    wait_edge(0)
    fire_gather(0, 0)

    def stage_group(k, _):
        for i in range(6):
            b = k * 6 + i

            @pl.when(b + 1 < nb)
            def _():
                wait_edge((i + 1) % 3)

            @pl.when((b >= 1) & (b <= nb))
            def _():
                wait_scatter((i + 1) % 2)

            @pl.when(b + 2 < nb)
            def _():
                fire_edge(b + 2, (i + 2) % 3)

            @pl.when(b < nb)
            def _():
                wait_gather(i % 2)

            @pl.when(b + 1 < nb)
            def _():
                fire_gather((i + 1) % 3, (i + 1) % 2)

            @pl.when(b < nb)
            def _():
                compute(i % 3, i % 2)
                fire_scatter(i % 3, i % 2)
        return 0
    lax.fori_loop(0, (nb + 1 + 5) // 6, stage_group, 0)
    plsc.subcore_barrier()

    sub_base = s * OUT_SUB
    rows_s = jnp.where(s == 15, OUT_SUB_LAST, OUT_SUB)

    def out_chunk(k, _):
        cs = jnp.minimum(k * CH, rows_s - CH)
        pltpu.sync_copy(acc_sh.at[pl.ds(sub_base + cs, CH)], r0)
        pltpu.sync_copy(inv_ref.at[pl.ds(coff + sub_base + cs, CH)], inv_v)

        def sr(r, _):
            iv = inv_v[r, :]
            r0[r, pl.ds(0, 16)] = r0[r, pl.ds(0, 16)] * iv
            r0[r, pl.ds(16, 16)] = r0[r, pl.ds(16, 16)] * iv
            return 0
        lax.fori_loop(0, CH, sr, 0)
        pltpu.sync_copy(r0, out_ref.at[pl.ds(coff + sub_base + cs, CH)])
        return 0
    lax.fori_loop(0, OUT_SUB // CH, out_chunk, 0)


def _run_hop(x, ed, inv):
    mesh = plsc.VectorSubcoreMesh(core_axis_name="c", subcore_axis_name="s")
    return pl.kernel(
        _hop_body,
        out_type=jax.ShapeDtypeStruct((N, H), f32),
        mesh=mesh,
        scratch_types=[
            pltpu.VMEM_SHARED((ACC_ROWS, H), f32),
            pltpu.VMEM((3, CH), i32),
            pltpu.VMEM((3, CH), i32),
            pltpu.VMEM((3, CH), i32),
            pltpu.VMEM((CH, H), f32),
            pltpu.VMEM((CH, H), f32),
            pltpu.VMEM((ZB, H), f32),
            pltpu.VMEM((CH, DEG_W), f32),
            pltpu.SemaphoreType.DMA,
            pltpu.SemaphoreType.DMA,
            pltpu.SemaphoreType.DMA,
            pltpu.SemaphoreType.DMA,
            pltpu.SemaphoreType.DMA,
            pltpu.SemaphoreType.DMA,
            pltpu.SemaphoreType.DMA,
        ],
        compiler_params=pltpu.CompilerParams(use_tc_tiling_on_sc=False,
                                             needs_layout_passes=False),
    )(x, ed, inv)


def _dec_body(h_ref, f1_ref, f2_ref, b1_ref, b2_ref, wdiff_ref, wdec_ref,
              bdiff_ref, bdec_ref, out_ref):
    wdec = wdec_ref[...]
    wf = jnp.dot(wdiff_ref[...], wdec, preferred_element_type=f32)
    bias = jnp.dot(bdiff_ref[...], wdec, preferred_element_type=f32) \
        + bdec_ref[...]
    acc = bias
    parts = (h_ref, f1_ref, f2_ref, b1_ref, b2_ref)
    for p, ref in enumerate(parts):
        acc = acc + jnp.dot(ref[...], wf[p * H:(p + 1) * H, :],
                            preferred_element_type=f32)
    out_ref[...] = acc


def _run_dec(h, f1, f2, b1, b2, W_diff, W_dec, b_diff, b_dec):
    NB = 5000
    grid = N // NB
    part_spec = pl.BlockSpec((NB, H), lambda i: (i, 0))
    return pl.pallas_call(
        _dec_body,
        grid=(grid,),
        in_specs=[
            part_spec, part_spec, part_spec, part_spec, part_spec,
            pl.BlockSpec((5 * H, H), lambda i: (0, 0)),
            pl.BlockSpec((H, HORIZON), lambda i: (0, 0)),
            pl.BlockSpec((1, H), lambda i: (0, 0)),
            pl.BlockSpec((1, HORIZON), lambda i: (0, 0)),
        ],
        out_specs=pl.BlockSpec((NB, HORIZON), lambda i: (i, 0)),
        out_shape=jax.ShapeDtypeStruct((N, HORIZON), f32),
    )(h, f1, f2, b1, b2, W_diff, W_dec, b_diff, b_dec)


def kernel(x, edge_index, edge_weight, W_enc, b_enc, node_emb, Wi, Wh, bi,
           bh, W_diff, b_diff, W_dec, b_dec):
    x2 = jnp.transpose(x.reshape(T, N))
    ei3 = edge_index.reshape(2, NBATCH, NCHUNK, CH)
    w3 = edge_weight.reshape(NBATCH, NCHUNK, CH)
    src2 = edge_index[0].reshape(NB2, 1, CH)
    dst2 = edge_index[1].reshape(NB2, 1, CH)
    wb = lax.bitcast_convert_type(edge_weight, i32).reshape(NB2, 1, CH)
    ed_f = jnp.concatenate([src2, dst2, wb], axis=1)
    ed_b = jnp.concatenate([dst2, src2, wb], axis=1)
    wrep = jnp.broadcast_to(edge_weight[:, None],
                            (E, DEG_W)).reshape(NB2, CH, DEG_W)

    h = _run_gru(x2, node_emb, W_enc, b_enc.reshape(1, H), Wi, Wh,
                 bi.reshape(1, 3 * H), bh.reshape(1, 3 * H))

    inv2 = _run_deg(ed_f, wrep)
    inv_f = inv2[:N]
    inv_b = inv2[N:]

    f1 = _run_hop(h, ed_f, inv_f)
    f2 = _run_hop(f1, ed_f, inv_f)
    b1 = _run_hop(h, ed_b, inv_b)
    b2 = _run_hop(b1, ed_b, inv_b)

    outn = _run_dec(h, f1, f2, b1, b2, W_diff, W_dec,
                    b_diff.reshape(1, H), b_dec.reshape(1, HORIZON))
    return jnp.transpose(outn.reshape(1, N, HORIZON, 1), (0, 2, 1, 3))


# speedup vs baseline: 1.2016x; 1.2016x over previous
import functools

import jax
import jax.numpy as jnp
from jax import lax
from jax.experimental import pallas as pl
from jax.experimental.pallas import tpu as pltpu
from jax.experimental.pallas import tpu_sc as plsc

N = 100000
E = 1600000
T = 12
H = 32
HORIZON = 12

NC = 2
NS = 16
HALF = N // 2

NBATCH = 6250
BE = 256
NCHUNK = 2
CH = 128

f32 = jnp.float32
i32 = jnp.int32


def _iota16():
    return lax.iota(i32, 16)


def _gru_body(x_ref, emb_ref, wenc_ref, benc_ref, wi_ref, wh_ref, bi_ref,
              bh_ref, out_ref):
    wi = wi_ref[...]
    wh = wh_ref[...]
    wv = jnp.dot(wenc_ref[...], wi, preferred_element_type=f32)
    c = jnp.dot(emb_ref[...] + benc_ref[...], wi,
                preferred_element_type=f32) + bi_ref[...]
    x = x_ref[...]
    nb = x.shape[0]
    h = jnp.zeros((nb, H), f32)
    for t in range(T):
        gi = x[:, t:t + 1] * wv + c
        gh = jnp.dot(h, wh, preferred_element_type=f32) + bh_ref[...]
        r = jax.nn.sigmoid(gi[:, :H] + gh[:, :H])
        z = jax.nn.sigmoid(gi[:, H:2 * H] + gh[:, H:2 * H])
        n = jnp.tanh(gi[:, 2 * H:] + r * gh[:, 2 * H:])
        h = (1.0 - z) * n + z * h
    out_ref[...] = h


def _run_gru(x2, node_emb, W_enc, b_enc, Wi, Wh, bi, bh):
    NB = 5000
    grid = N // NB
    return pl.pallas_call(
        _gru_body,
        grid=(grid,),
        in_specs=[
            pl.BlockSpec((NB, T), lambda i: (i, 0)),
            pl.BlockSpec((NB, H), lambda i: (i, 0)),
            pl.BlockSpec((1, H), lambda i: (0, 0)),
            pl.BlockSpec((1, H), lambda i: (0, 0)),
            pl.BlockSpec((H, 3 * H), lambda i: (0, 0)),
            pl.BlockSpec((H, 3 * H), lambda i: (0, 0)),
            pl.BlockSpec((1, 3 * H), lambda i: (0, 0)),
            pl.BlockSpec((1, 3 * H), lambda i: (0, 0)),
        ],
        out_specs=pl.BlockSpec((NB, H), lambda i: (i, 0)),
        out_shape=jax.ShapeDtypeStruct((N, H), f32),
    )(x2, node_emb, W_enc, b_enc, Wi, Wh, bi, bh)


DEG_W = 16
DEG_SUB = 6400
DEG_SUB_LAST = N - 15 * DEG_SUB
DEG_CHUNK = 200


def _deg_body(ed_ref, inv_ref, acc_sh, e0, e1, e2, w0, w1, w2,
              zbuf, dbuf, se0, se1, se2, ss0, ss1):
    c = lax.axis_index("c")
    s = lax.axis_index("s")
    zeros16f = jnp.zeros((16,), f32)
    srow = 1 - c
    ebufs, esems = [e0, e1, e2], [se0, se1, se2]
    wbufs = [w0, w1, w2]
    ssems = [ss0, ss1]

    def zb(i, _):
        zbuf[i, :] = zeros16f
        return 0
    lax.fori_loop(0, ZB, zb, 0)
    zbase = s * DEG_SUB
    zrows = jnp.where(s == 15, DEG_SUB_LAST, DEG_SUB)

    def zc(k, _):
        off = jnp.minimum(k * ZB, zrows - ZB)
        pltpu.sync_copy(zbuf, acc_sh.at[pl.ds(zbase + off, ZB)])
        return 0
    lax.fori_loop(0, DEG_SUB // ZB, zc, 0)
    plsc.subcore_barrier()

    nb = 781 + jnp.where(s < 4, 1, 0)
    start_s = s * 781 + jnp.minimum(s, 4)

    def fire_edge(b, ei):
        pltpu.async_copy(ed_ref.at[start_s + b], ebufs[ei], esems[ei])

    def wait_edge(ei):
        pltpu.make_async_copy(ed_ref.at[0], ebufs[ei], esems[ei]).wait()

    def fill_msg(ei):
        es, ws = ebufs[ei], wbufs[ei]

        def fg(g, _):
            w16 = plsc.bitcast(es[2, pl.ds(g * 16, 16)], f32)
            for i in range(16):
                ws[g * 16 + i, :] = jnp.full((16,), w16[i], f32)
            return 0
        lax.fori_loop(0, 8, fg, 0)

    def fire_scatter(ei, ri):
        pltpu.async_copy(wbufs[ei], acc_sh.at[ebufs[ei].at[srow]],
                         ssems[ri], add=True)

    def wait_scatter(ei, ri):
        pltpu.make_async_copy(wbufs[ei], acc_sh.at[pl.ds(0, CH)],
                              ssems[ri]).wait()

    fire_edge(0, 0)
    fire_edge(1, 1)

    def stage_group(k, _):
        for i in range(6):
            b = k * 6 + i

            @pl.when((b >= 1) & (b <= nb))
            def _():
                wait_scatter((i + 2) % 3, (i + 1) % 2)

            @pl.when(b + 2 < nb)
            def _():
                fire_edge(b + 2, (i + 2) % 3)

            @pl.when(b < nb)
            def _():
                wait_edge(i % 3)
                fill_msg(i % 3)
                fire_scatter(i % 3, i % 2)
        return 0
    lax.fori_loop(0, (nb + 1 + 5) // 6, stage_group, 0)
    plsc.subcore_barrier()

    base = s * DEG_SUB
    for k in range(DEG_SUB // DEG_CHUNK):
        off = base + jnp.minimum(k * DEG_CHUNK, zrows - DEG_CHUNK)
        pltpu.sync_copy(acc_sh.at[pl.ds(off, DEG_CHUNK)], dbuf)

        def ir(r, _):
            deg = dbuf[r, :]
            dbuf[r, :] = 1.0 / jnp.where(deg == 0.0, 1.0, deg)
            return 0
        lax.fori_loop(0, DEG_CHUNK, ir, 0)
        pltpu.sync_copy(dbuf, inv_ref.at[pl.ds(c * N + off, DEG_CHUNK)])


def _run_deg(ed):
    mesh = plsc.VectorSubcoreMesh(core_axis_name="c", subcore_axis_name="s")
    return pl.kernel(
        _deg_body,
        out_type=jax.ShapeDtypeStruct((2 * N, DEG_W), f32),
        mesh=mesh,
        scratch_types=[
            pltpu.VMEM_SHARED((N, DEG_W), f32),
            pltpu.VMEM((3, CH), i32),
            pltpu.VMEM((3, CH), i32),
            pltpu.VMEM((3, CH), i32),
            pltpu.VMEM((CH, DEG_W), f32),
            pltpu.VMEM((CH, DEG_W), f32),
            pltpu.VMEM((CH, DEG_W), f32),
            pltpu.VMEM((ZB, DEG_W), f32),
            pltpu.VMEM((DEG_CHUNK, DEG_W), f32),
            pltpu.SemaphoreType.DMA,
            pltpu.SemaphoreType.DMA,
            pltpu.SemaphoreType.DMA,
            pltpu.SemaphoreType.DMA,
            pltpu.SemaphoreType.DMA,
        ],
        compiler_params=pltpu.CompilerParams(use_tc_tiling_on_sc=False,
                                             needs_layout_passes=False),
    )(ed)


ACC_ROWS = 50176
OUT_SUB = 3200
OUT_SUB_LAST = HALF - 15 * OUT_SUB
ZB = 16
NB2 = E // CH


def _hop_body(x_ref, ed_ref, inv_ref, out_ref, acc_sh,
              e0, e1, e2, r0, r1, zbuf, inv_v,
              se0, se1, se2, sg0, sg1, ss0, ss1):
    c = lax.axis_index("c")
    s = lax.axis_index("s")
    iota = _iota16()
    zeros16f = jnp.zeros((16,), f32)
    coff = c * HALF
    ebufs, esems = [e0, e1, e2], [se0, se1, se2]
    rbufs, gsems, ssems = [r0, r1], [sg0, sg1], [ss0, ss1]

    def zb(i, _):
        zbuf[i, pl.ds(0, 16)] = zeros16f
        zbuf[i, pl.ds(16, 16)] = zeros16f
        return 0
    lax.fori_loop(0, ZB, zb, 0)
    zrows = ACC_ROWS // NS

    def zc(k, _):
        pltpu.sync_copy(zbuf, acc_sh.at[pl.ds(s * zrows + k * ZB, ZB)])
        return 0
    lax.fori_loop(0, zrows // ZB, zc, 0)
    plsc.subcore_barrier()

    nb = 781 + jnp.where(s < 4, 1, 0)
    start_s = s * 781 + jnp.minimum(s, 4)

    def fire_edge(b, ei):
        pltpu.async_copy(ed_ref.at[start_s + b], ebufs[ei], esems[ei])

    def wait_edge(ei):
        pltpu.make_async_copy(ed_ref.at[0], ebufs[ei], esems[ei]).wait()

    def fire_gather(ei, ri):
        pltpu.async_copy(x_ref.at[ebufs[ei].at[0]], rbufs[ri], gsems[ri])

    def wait_gather(ri):
        pltpu.make_async_copy(x_ref.at[pl.ds(0, CH)], rbufs[ri],
                              gsems[ri]).wait()

    def fire_scatter(ei, ri):
        pltpu.async_copy(rbufs[ri], acc_sh.at[ebufs[ei].at[1]], ssems[ri],
                         add=True)

    def wait_scatter(ri):
        pltpu.make_async_copy(rbufs[ri], acc_sh.at[pl.ds(0, CH)],
                              ssems[ri]).wait()

    def compute(ei, ri):
        es, rs = ebufs[ei], rbufs[ri]

        def wm(g, _):
            sl = pl.ds(g * 16, 16)
            dst = es[1, sl]
            local = dst - coff
            ok = (local >= 0) & (local < HALF)
            trash = HALF + ((s * 16 + g * 16 + iota) & 127)
            es[1, sl] = jnp.where(ok, local, trash)
            w16 = plsc.bitcast(es[2, sl], f32)
            for i in range(16):
                wsp = jnp.full((16,), w16[i], f32)
                r = g * 16 + i
                rs[r, pl.ds(0, 16)] = rs[r, pl.ds(0, 16)] * wsp
                rs[r, pl.ds(16, 16)] = rs[r, pl.ds(16, 16)] * wsp
            return 0
        lax.fori_loop(0, 8, wm, 0)

    fire_edge(0, 0)
    fire_edge(1, 1)
    wait_edge(0)
    fire_gather(0, 0)

    def stage_group(k, _):
        for i in range(6):
            b = k * 6 + i

            @pl.when(b + 1 < nb)
            def _():
                wait_edge((i + 1) % 3)

            @pl.when((b >= 1) & (b <= nb))
            def _():
                wait_scatter((i + 1) % 2)

            @pl.when(b + 2 < nb)
            def _():
                fire_edge(b + 2, (i + 2) % 3)

            @pl.when(b < nb)
            def _():
                wait_gather(i % 2)

            @pl.when(b + 1 < nb)
            def _():
                fire_gather((i + 1) % 3, (i + 1) % 2)

            @pl.when(b < nb)
            def _():
                compute(i % 3, i % 2)
                fire_scatter(i % 3, i % 2)
        return 0
    lax.fori_loop(0, (nb + 1 + 5) // 6, stage_group, 0)
    plsc.subcore_barrier()

    sub_base = s * OUT_SUB
    rows_s = jnp.where(s == 15, OUT_SUB_LAST, OUT_SUB)

    def out_chunk(k, _):
        cs = jnp.minimum(k * CH, rows_s - CH)
        pltpu.sync_copy(acc_sh.at[pl.ds(sub_base + cs, CH)], r0)
        pltpu.sync_copy(inv_ref.at[pl.ds(coff + sub_base + cs, CH)], inv_v)

        def sr(r, _):
            iv = inv_v[r, :]
            r0[r, pl.ds(0, 16)] = r0[r, pl.ds(0, 16)] * iv
            r0[r, pl.ds(16, 16)] = r0[r, pl.ds(16, 16)] * iv
            return 0
        lax.fori_loop(0, CH, sr, 0)
        pltpu.sync_copy(r0, out_ref.at[pl.ds(coff + sub_base + cs, CH)])
        return 0
    lax.fori_loop(0, OUT_SUB // CH, out_chunk, 0)


def _run_hop(x, ed, inv):
    mesh = plsc.VectorSubcoreMesh(core_axis_name="c", subcore_axis_name="s")
    return pl.kernel(
        _hop_body,
        out_type=jax.ShapeDtypeStruct((N, H), f32),
        mesh=mesh,
        scratch_types=[
            pltpu.VMEM_SHARED((ACC_ROWS, H), f32),
            pltpu.VMEM((3, CH), i32),
            pltpu.VMEM((3, CH), i32),
            pltpu.VMEM((3, CH), i32),
            pltpu.VMEM((CH, H), f32),
            pltpu.VMEM((CH, H), f32),
            pltpu.VMEM((ZB, H), f32),
            pltpu.VMEM((CH, DEG_W), f32),
            pltpu.SemaphoreType.DMA,
            pltpu.SemaphoreType.DMA,
            pltpu.SemaphoreType.DMA,
            pltpu.SemaphoreType.DMA,
            pltpu.SemaphoreType.DMA,
            pltpu.SemaphoreType.DMA,
            pltpu.SemaphoreType.DMA,
        ],
        compiler_params=pltpu.CompilerParams(use_tc_tiling_on_sc=False,
                                             needs_layout_passes=False),
    )(x, ed, inv)


def _dec_body(h_ref, f1_ref, f2_ref, b1_ref, b2_ref, wdiff_ref, wdec_ref,
              bdiff_ref, bdec_ref, out_ref):
    wdec = wdec_ref[...]
    wf = jnp.dot(wdiff_ref[...], wdec, preferred_element_type=f32)
    bias = jnp.dot(bdiff_ref[...], wdec, preferred_element_type=f32) \
        + bdec_ref[...]
    acc = bias
    parts = (h_ref, f1_ref, f2_ref, b1_ref, b2_ref)
    for p, ref in enumerate(parts):
        acc = acc + jnp.dot(ref[...], wf[p * H:(p + 1) * H, :],
                            preferred_element_type=f32)
    out_ref[...] = acc


def _run_dec(h, f1, f2, b1, b2, W_diff, W_dec, b_diff, b_dec):
    NB = 5000
    grid = N // NB
    part_spec = pl.BlockSpec((NB, H), lambda i: (i, 0))
    return pl.pallas_call(
        _dec_body,
        grid=(grid,),
        in_specs=[
            part_spec, part_spec, part_spec, part_spec, part_spec,
            pl.BlockSpec((5 * H, H), lambda i: (0, 0)),
            pl.BlockSpec((H, HORIZON), lambda i: (0, 0)),
            pl.BlockSpec((1, H), lambda i: (0, 0)),
            pl.BlockSpec((1, HORIZON), lambda i: (0, 0)),
        ],
        out_specs=pl.BlockSpec((NB, HORIZON), lambda i: (i, 0)),
        out_shape=jax.ShapeDtypeStruct((N, HORIZON), f32),
    )(h, f1, f2, b1, b2, W_diff, W_dec, b_diff, b_dec)


def kernel(x, edge_index, edge_weight, W_enc, b_enc, node_emb, Wi, Wh, bi,
           bh, W_diff, b_diff, W_dec, b_dec):
    x2 = jnp.transpose(x.reshape(T, N))
    ei3 = edge_index.reshape(2, NBATCH, NCHUNK, CH)
    w3 = edge_weight.reshape(NBATCH, NCHUNK, CH)
    src2 = edge_index[0].reshape(NB2, 1, CH)
    dst2 = edge_index[1].reshape(NB2, 1, CH)
    wb = lax.bitcast_convert_type(edge_weight, i32).reshape(NB2, 1, CH)
    ed_f = jnp.concatenate([src2, dst2, wb], axis=1)
    ed_b = jnp.concatenate([dst2, src2, wb], axis=1)

    h = _run_gru(x2, node_emb, W_enc, b_enc.reshape(1, H), Wi, Wh,
                 bi.reshape(1, 3 * H), bh.reshape(1, 3 * H))

    inv2 = _run_deg(ed_f)
    inv_f = inv2[:N]
    inv_b = inv2[N:]

    f1 = _run_hop(h, ed_f, inv_f)
    f2 = _run_hop(f1, ed_f, inv_f)
    b1 = _run_hop(h, ed_b, inv_b)
    b2 = _run_hop(b1, ed_b, inv_b)

    outn = _run_dec(h, f1, f2, b1, b2, W_diff, W_dec,
                    b_diff.reshape(1, H), b_dec.reshape(1, HORIZON))
    return jnp.transpose(outn.reshape(1, N, HORIZON, 1), (0, 2, 1, 3))


# speedup vs baseline: 1.2022x; 1.0005x over previous
import functools

import jax
import jax.numpy as jnp
from jax import lax
from jax.experimental import pallas as pl
from jax.experimental.pallas import tpu as pltpu
from jax.experimental.pallas import tpu_sc as plsc

N = 100000
E = 1600000
T = 12
H = 32
HORIZON = 12

NC = 2
NS = 16
HALF = N // 2

NBATCH = 6250
BE = 256
NCHUNK = 2
CH = 128

f32 = jnp.float32
i32 = jnp.int32


def _iota16():
    return lax.iota(i32, 16)


def _gru_body(x_ref, emb_ref, wenc_ref, benc_ref, wi_ref, wh_ref, bi_ref,
              bh_ref, out_ref):
    wi = wi_ref[...]
    wh = wh_ref[...]
    wv = jnp.dot(wenc_ref[...], wi, preferred_element_type=f32)
    c = jnp.dot(emb_ref[...] + benc_ref[...], wi,
                preferred_element_type=f32) + bi_ref[...]
    x = x_ref[...]
    nb = x.shape[0]
    h = jnp.zeros((nb, H), f32)
    for t in range(T):
        gi = x[:, t:t + 1] * wv + c
        gh = jnp.dot(h, wh, preferred_element_type=f32) + bh_ref[...]
        r = jax.nn.sigmoid(gi[:, :H] + gh[:, :H])
        z = jax.nn.sigmoid(gi[:, H:2 * H] + gh[:, H:2 * H])
        n = jnp.tanh(gi[:, 2 * H:] + r * gh[:, 2 * H:])
        h = (1.0 - z) * n + z * h
    out_ref[...] = h


def _run_gru(x2, node_emb, W_enc, b_enc, Wi, Wh, bi, bh):
    NB = 5000
    grid = N // NB
    return pl.pallas_call(
        _gru_body,
        grid=(grid,),
        in_specs=[
            pl.BlockSpec((NB, T), lambda i: (i, 0)),
            pl.BlockSpec((NB, H), lambda i: (i, 0)),
            pl.BlockSpec((1, H), lambda i: (0, 0)),
            pl.BlockSpec((1, H), lambda i: (0, 0)),
            pl.BlockSpec((H, 3 * H), lambda i: (0, 0)),
            pl.BlockSpec((H, 3 * H), lambda i: (0, 0)),
            pl.BlockSpec((1, 3 * H), lambda i: (0, 0)),
            pl.BlockSpec((1, 3 * H), lambda i: (0, 0)),
        ],
        out_specs=pl.BlockSpec((NB, H), lambda i: (i, 0)),
        out_shape=jax.ShapeDtypeStruct((N, H), f32),
    )(x2, node_emb, W_enc, b_enc, Wi, Wh, bi, bh)


DEG_W = 16
DEG_SUB = 6400
DEG_SUB_LAST = N - 15 * DEG_SUB
DEG_CHUNK = 200


def _deg_body(ed_ref, inv_ref, acc_sh, e0, e1, e2, w0, w1, w2,
              zbuf, dbuf, se0, se1, se2, ss0, ss1):
    c = lax.axis_index("c")
    s = lax.axis_index("s")
    zeros16f = jnp.zeros((16,), f32)
    srow = 1 - c
    ebufs, esems = [e0, e1, e2], [se0, se1, se2]
    wbufs = [w0, w1, w2]
    ssems = [ss0, ss1]

    def zb(i, _):
        zbuf[i, :] = zeros16f
        return 0
    lax.fori_loop(0, ZB, zb, 0)
    zbase = s * DEG_SUB
    zrows = jnp.where(s == 15, DEG_SUB_LAST, DEG_SUB)

    def zc(k, _):
        off = jnp.minimum(k * ZB, zrows - ZB)
        pltpu.sync_copy(zbuf, acc_sh.at[pl.ds(zbase + off, ZB)])
        return 0
    lax.fori_loop(0, DEG_SUB // ZB, zc, 0)
    plsc.subcore_barrier()

    nb = 781 + jnp.where(s < 4, 1, 0)
    start_s = s * 781 + jnp.minimum(s, 4)

    def fire_edge(b, ei):
        pltpu.async_copy(ed_ref.at[start_s + b], ebufs[ei], esems[ei])

    def wait_edge(ei):
        pltpu.make_async_copy(ed_ref.at[0], ebufs[ei], esems[ei]).wait()

    def fill_msg(ei):
        es, ws = ebufs[ei], wbufs[ei]

        def fg(g, _):
            w16 = plsc.bitcast(es[2, pl.ds(g * 16, 16)], f32)
            for i in range(16):
                ws[g * 16 + i, :] = jnp.full((16,), w16[i], f32)
            return 0
        lax.fori_loop(0, 8, fg, 0)

    def fire_scatter(ei, ri):
        pltpu.async_copy(wbufs[ei], acc_sh.at[ebufs[ei].at[srow]],
                         ssems[ri], add=True)

    def wait_scatter(ei, ri):
        pltpu.make_async_copy(wbufs[ei], acc_sh.at[pl.ds(0, CH)],
                              ssems[ri]).wait()

    fire_edge(0, 0)
    fire_edge(1, 1)

    def stage_group(k, _):
        for i in range(6):
            b = k * 6 + i

            @pl.when((b >= 1) & (b <= nb))
            def _():
                wait_scatter((i + 2) % 3, (i + 1) % 2)

            @pl.when(b + 2 < nb)
            def _():
                fire_edge(b + 2, (i + 2) % 3)

            @pl.when(b < nb)
            def _():
                wait_edge(i % 3)
                fill_msg(i % 3)
                fire_scatter(i % 3, i % 2)
        return 0
    lax.fori_loop(0, (nb + 1 + 5) // 6, stage_group, 0)
    plsc.subcore_barrier()

    base = s * DEG_SUB
    for k in range(DEG_SUB // DEG_CHUNK):
        off = base + jnp.minimum(k * DEG_CHUNK, zrows - DEG_CHUNK)
        pltpu.sync_copy(acc_sh.at[pl.ds(off, DEG_CHUNK)], dbuf)

        def ir(r, _):
            deg = dbuf[r, :]
            dbuf[r, :] = 1.0 / jnp.where(deg == 0.0, 1.0, deg)
            return 0
        lax.fori_loop(0, DEG_CHUNK, ir, 0)
        pltpu.sync_copy(dbuf, inv_ref.at[pl.ds(c * N + off, DEG_CHUNK)])


def _run_deg(ed):
    mesh = plsc.VectorSubcoreMesh(core_axis_name="c", subcore_axis_name="s")
    return pl.kernel(
        _deg_body,
        out_type=jax.ShapeDtypeStruct((2 * N, DEG_W), f32),
        mesh=mesh,
        scratch_types=[
            pltpu.VMEM_SHARED((N, DEG_W), f32),
            pltpu.VMEM((3, CH), i32),
            pltpu.VMEM((3, CH), i32),
            pltpu.VMEM((3, CH), i32),
            pltpu.VMEM((CH, DEG_W), f32),
            pltpu.VMEM((CH, DEG_W), f32),
            pltpu.VMEM((CH, DEG_W), f32),
            pltpu.VMEM((ZB, DEG_W), f32),
            pltpu.VMEM((DEG_CHUNK, DEG_W), f32),
            pltpu.SemaphoreType.DMA,
            pltpu.SemaphoreType.DMA,
            pltpu.SemaphoreType.DMA,
            pltpu.SemaphoreType.DMA,
            pltpu.SemaphoreType.DMA,
        ],
        compiler_params=pltpu.CompilerParams(use_tc_tiling_on_sc=False,
                                             needs_layout_passes=False),
    )(ed)


ACC_ROWS = 51200
OUT_SUB = 3200
OUT_SUB_LAST = HALF - 15 * OUT_SUB
ZB = 16
NB2 = E // CH


def _hop_body(x_ref, ed_ref, inv_ref, out_ref, acc_sh,
              e0, e1, e2, r0, r1, zbuf, inv_v,
              se0, se1, se2, sg0, sg1, ss0, ss1):
    c = lax.axis_index("c")
    s = lax.axis_index("s")
    iota = _iota16()
    zeros16f = jnp.zeros((16,), f32)
    coff = c * HALF
    ebufs, esems = [e0, e1, e2], [se0, se1, se2]
    rbufs, gsems, ssems = [r0, r1], [sg0, sg1], [ss0, ss1]

    def zb(i, _):
        zbuf[i, pl.ds(0, 16)] = zeros16f
        zbuf[i, pl.ds(16, 16)] = zeros16f
        return 0
    lax.fori_loop(0, ZB, zb, 0)
    zrows = ACC_ROWS // NS

    def zc(k, _):
        pltpu.sync_copy(zbuf, acc_sh.at[pl.ds(s * zrows + k * ZB, ZB)])
        return 0
    lax.fori_loop(0, zrows // ZB, zc, 0)
    plsc.subcore_barrier()

    nb = 781 + jnp.where(s < 4, 1, 0)
    start_s = s * 781 + jnp.minimum(s, 4)

    def fire_edge(b, ei):
        pltpu.async_copy(ed_ref.at[start_s + b], ebufs[ei], esems[ei])

    def wait_edge(ei):
        pltpu.make_async_copy(ed_ref.at[0], ebufs[ei], esems[ei]).wait()

    def fire_gather(ei, ri):
        pltpu.async_copy(x_ref.at[ebufs[ei].at[0]], rbufs[ri], gsems[ri])

    def wait_gather(ri):
        pltpu.make_async_copy(x_ref.at[pl.ds(0, CH)], rbufs[ri],
                              gsems[ri]).wait()

    def fire_scatter(ei, ri):
        pltpu.async_copy(rbufs[ri], acc_sh.at[ebufs[ei].at[1]], ssems[ri],
                         add=True)

    def wait_scatter(ri):
        pltpu.make_async_copy(rbufs[ri], acc_sh.at[pl.ds(0, CH)],
                              ssems[ri]).wait()

    def compute(ei, ri):
        es, rs = ebufs[ei], rbufs[ri]

        def wm(g, _):
            sl = pl.ds(g * 16, 16)
            dst = es[1, sl]
            local = dst - coff
            ok = (local >= 0) & (local < HALF)
            trash = HALF + s * 64 + ((g * 16 + iota) & 63)
            es[1, sl] = jnp.where(ok, local, trash)
            w16 = plsc.bitcast(es[2, sl], f32)
            for i in range(16):
                wsp = jnp.full((16,), w16[i], f32)
                r = g * 16 + i
                rs[r, pl.ds(0, 16)] = rs[r, pl.ds(0, 16)] * wsp
                rs[r, pl.ds(16, 16)] = rs[r, pl.ds(16, 16)] * wsp
            return 0
        lax.fori_loop(0, 8, wm, 0)

    fire_edge(0, 0)
    fire_edge(1, 1)
    wait_edge(0)
    fire_gather(0, 0)

    def stage_group(k, _):
        for i in range(6):
            b = k * 6 + i

            @pl.when(b + 1 < nb)
            def _():
                wait_edge((i + 1) % 3)

            @pl.when((b >= 1) & (b <= nb))
            def _():
                wait_scatter((i + 1) % 2)

            @pl.when(b + 2 < nb)
            def _():
                fire_edge(b + 2, (i + 2) % 3)

            @pl.when(b < nb)
            def _():
                wait_gather(i % 2)

            @pl.when(b + 1 < nb)
            def _():
                fire_gather((i + 1) % 3, (i + 1) % 2)

            @pl.when(b < nb)
            def _():
                compute(i % 3, i % 2)
                fire_scatter(i % 3, i % 2)
        return 0
    lax.fori_loop(0, (nb + 1 + 5) // 6, stage_group, 0)
    plsc.subcore_barrier()

    sub_base = s * OUT_SUB
    rows_s = jnp.where(s == 15, OUT_SUB_LAST, OUT_SUB)

    def out_chunk(k, _):
        cs = jnp.minimum(k * CH, rows_s - CH)
        pltpu.sync_copy(acc_sh.at[pl.ds(sub_base + cs, CH)], r0)
        pltpu.sync_copy(inv_ref.at[pl.ds(coff + sub_base + cs, CH)], inv_v)

        def sr(r, _):
            iv = inv_v[r, :]
            r0[r, pl.ds(0, 16)] = r0[r, pl.ds(0, 16)] * iv
            r0[r, pl.ds(16, 16)] = r0[r, pl.ds(16, 16)] * iv
            return 0
        lax.fori_loop(0, CH, sr, 0)
        pltpu.sync_copy(r0, out_ref.at[pl.ds(coff + sub_base + cs, CH)])
        return 0
    lax.fori_loop(0, OUT_SUB // CH, out_chunk, 0)


def _run_hop(x, ed, inv):
    mesh = plsc.VectorSubcoreMesh(core_axis_name="c", subcore_axis_name="s")
    return pl.kernel(
        _hop_body,
        out_type=jax.ShapeDtypeStruct((N, H), f32),
        mesh=mesh,
        scratch_types=[
            pltpu.VMEM_SHARED((ACC_ROWS, H), f32),
            pltpu.VMEM((3, CH), i32),
            pltpu.VMEM((3, CH), i32),
            pltpu.VMEM((3, CH), i32),
            pltpu.VMEM((CH, H), f32),
            pltpu.VMEM((CH, H), f32),
            pltpu.VMEM((ZB, H), f32),
            pltpu.VMEM((CH, DEG_W), f32),
            pltpu.SemaphoreType.DMA,
            pltpu.SemaphoreType.DMA,
            pltpu.SemaphoreType.DMA,
            pltpu.SemaphoreType.DMA,
            pltpu.SemaphoreType.DMA,
            pltpu.SemaphoreType.DMA,
            pltpu.SemaphoreType.DMA,
        ],
        compiler_params=pltpu.CompilerParams(use_tc_tiling_on_sc=False,
                                             needs_layout_passes=False),
    )(x, ed, inv)


def _dec_body(h_ref, f1_ref, f2_ref, b1_ref, b2_ref, wdiff_ref, wdec_ref,
              bdiff_ref, bdec_ref, out_ref):
    wdec = wdec_ref[...]
    wf = jnp.dot(wdiff_ref[...], wdec, preferred_element_type=f32)
    bias = jnp.dot(bdiff_ref[...], wdec, preferred_element_type=f32) \
        + bdec_ref[...]
    acc = bias
    parts = (h_ref, f1_ref, f2_ref, b1_ref, b2_ref)
    for p, ref in enumerate(parts):
        acc = acc + jnp.dot(ref[...], wf[p * H:(p + 1) * H, :],
                            preferred_element_type=f32)
    out_ref[...] = acc


def _run_dec(h, f1, f2, b1, b2, W_diff, W_dec, b_diff, b_dec):
    NB = 5000
    grid = N // NB
    part_spec = pl.BlockSpec((NB, H), lambda i: (i, 0))
    return pl.pallas_call(
        _dec_body,
        grid=(grid,),
        in_specs=[
            part_spec, part_spec, part_spec, part_spec, part_spec,
            pl.BlockSpec((5 * H, H), lambda i: (0, 0)),
            pl.BlockSpec((H, HORIZON), lambda i: (0, 0)),
            pl.BlockSpec((1, H), lambda i: (0, 0)),
            pl.BlockSpec((1, HORIZON), lambda i: (0, 0)),
        ],
        out_specs=pl.BlockSpec((NB, HORIZON), lambda i: (i, 0)),
        out_shape=jax.ShapeDtypeStruct((N, HORIZON), f32),
    )(h, f1, f2, b1, b2, W_diff, W_dec, b_diff, b_dec)


def kernel(x, edge_index, edge_weight, W_enc, b_enc, node_emb, Wi, Wh, bi,
           bh, W_diff, b_diff, W_dec, b_dec):
    x2 = jnp.transpose(x.reshape(T, N))
    ei3 = edge_index.reshape(2, NBATCH, NCHUNK, CH)
    w3 = edge_weight.reshape(NBATCH, NCHUNK, CH)
    src2 = edge_index[0].reshape(NB2, 1, CH)
    dst2 = edge_index[1].reshape(NB2, 1, CH)
    wb = lax.bitcast_convert_type(edge_weight, i32).reshape(NB2, 1, CH)
    ed_f = jnp.concatenate([src2, dst2, wb], axis=1)
    ed_b = jnp.concatenate([dst2, src2, wb], axis=1)

    h = _run_gru(x2, node_emb, W_enc, b_enc.reshape(1, H), Wi, Wh,
                 bi.reshape(1, 3 * H), bh.reshape(1, 3 * H))

    inv2 = _run_deg(ed_f)
    inv_f = inv2[:N]
    inv_b = inv2[N:]

    f1 = _run_hop(h, ed_f, inv_f)
    f2 = _run_hop(f1, ed_f, inv_f)
    b1 = _run_hop(h, ed_b, inv_b)
    b2 = _run_hop(b1, ed_b, inv_b)

    outn = _run_dec(h, f1, f2, b1, b2, W_diff, W_dec,
                    b_diff.reshape(1, H), b_dec.reshape(1, HORIZON))
    return jnp.transpose(outn.reshape(1, N, HORIZON, 1), (0, 2, 1, 3))


# speedup vs baseline: 1.2023x; 1.0001x over previous
import functools

import jax
import jax.numpy as jnp
from jax import lax
from jax.experimental import pallas as pl
from jax.experimental.pallas import tpu as pltpu
from jax.experimental.pallas import tpu_sc as plsc

N = 100000
E = 1600000
T = 12
H = 32
HORIZON = 12

NC = 2
NS = 16
HALF = N // 2

CH = 128

f32 = jnp.float32
i32 = jnp.int32


def _iota16():
    return lax.iota(i32, 16)


def _gru_body(x_ref, emb_ref, wenc_ref, benc_ref, wi_ref, wh_ref, bi_ref,
              bh_ref, out_ref):
    wi = wi_ref[...]
    wh = wh_ref[...]
    wv = jnp.dot(wenc_ref[...], wi, preferred_element_type=f32)
    c = jnp.dot(emb_ref[...] + benc_ref[...], wi,
                preferred_element_type=f32) + bi_ref[...]
    x = x_ref[...]
    nb = x.shape[0]
    h = jnp.zeros((nb, H), f32)
    for t in range(T):
        gi = x[:, t:t + 1] * wv + c
        gh = jnp.dot(h, wh, preferred_element_type=f32) + bh_ref[...]
        r = jax.nn.sigmoid(gi[:, :H] + gh[:, :H])
        z = jax.nn.sigmoid(gi[:, H:2 * H] + gh[:, H:2 * H])
        n = jnp.tanh(gi[:, 2 * H:] + r * gh[:, 2 * H:])
        h = (1.0 - z) * n + z * h
    out_ref[...] = h


def _run_gru(x2, node_emb, W_enc, b_enc, Wi, Wh, bi, bh):
    NB = 5000
    grid = N // NB
    return pl.pallas_call(
        _gru_body,
        grid=(grid,),
        in_specs=[
            pl.BlockSpec((NB, T), lambda i: (i, 0)),
            pl.BlockSpec((NB, H), lambda i: (i, 0)),
            pl.BlockSpec((1, H), lambda i: (0, 0)),
            pl.BlockSpec((1, H), lambda i: (0, 0)),
            pl.BlockSpec((H, 3 * H), lambda i: (0, 0)),
            pl.BlockSpec((H, 3 * H), lambda i: (0, 0)),
            pl.BlockSpec((1, 3 * H), lambda i: (0, 0)),
            pl.BlockSpec((1, 3 * H), lambda i: (0, 0)),
        ],
        out_specs=pl.BlockSpec((NB, H), lambda i: (i, 0)),
        out_shape=jax.ShapeDtypeStruct((N, H), f32),
    )(x2, node_emb, W_enc, b_enc, Wi, Wh, bi, bh)


DEG_W = 16
DEG_SUB = 6400
DEG_SUB_LAST = N - 15 * DEG_SUB
DEG_CHUNK = 200


def _deg_body(ed_ref, inv_ref, acc_sh, e0, e1, e2, w0, w1, w2,
              zbuf, dbuf, se0, se1, se2, ss0, ss1):
    c = lax.axis_index("c")
    s = lax.axis_index("s")
    zeros16f = jnp.zeros((16,), f32)
    srow = 1 - c
    ebufs, esems = [e0, e1, e2], [se0, se1, se2]
    wbufs = [w0, w1, w2]
    ssems = [ss0, ss1]

    def zb(i, _):
        zbuf[i, :] = zeros16f
        return 0
    lax.fori_loop(0, ZB, zb, 0)
    zbase = s * DEG_SUB
    zrows = jnp.where(s == 15, DEG_SUB_LAST, DEG_SUB)

    def zc(k, _):
        off = jnp.minimum(k * ZB, zrows - ZB)
        pltpu.sync_copy(zbuf, acc_sh.at[pl.ds(zbase + off, ZB)])
        return 0
    lax.fori_loop(0, DEG_SUB // ZB, zc, 0)
    plsc.subcore_barrier()

    nb = 781 + jnp.where(s < 4, 1, 0)
    start_s = s * 781 + jnp.minimum(s, 4)

    def fire_edge(b, ei):
        pltpu.async_copy(ed_ref.at[start_s + b], ebufs[ei], esems[ei])

    def wait_edge(ei):
        pltpu.make_async_copy(ed_ref.at[0], ebufs[ei], esems[ei]).wait()

    def fill_msg(ei):
        es, ws = ebufs[ei], wbufs[ei]

        def fg(g, _):
            w16 = plsc.bitcast(es[2, pl.ds(g * 16, 16)], f32)
            for i in range(16):
                ws[g * 16 + i, :] = jnp.full((16,), w16[i], f32)
            return 0
        lax.fori_loop(0, 8, fg, 0)

    def fire_scatter(ei, ri):
        pltpu.async_copy(wbufs[ei], acc_sh.at[ebufs[ei].at[srow]],
                         ssems[ri], add=True)

    def wait_scatter(ei, ri):
        pltpu.make_async_copy(wbufs[ei], acc_sh.at[pl.ds(0, CH)],
                              ssems[ri]).wait()

    fire_edge(0, 0)
    fire_edge(1, 1)

    def stage_group(k, _):
        for i in range(6):
            b = k * 6 + i

            @pl.when((b >= 1) & (b <= nb))
            def _():
                wait_scatter((i + 2) % 3, (i + 1) % 2)

            @pl.when(b + 2 < nb)
            def _():
                fire_edge(b + 2, (i + 2) % 3)

            @pl.when(b < nb)
            def _():
                wait_edge(i % 3)
                fill_msg(i % 3)
                fire_scatter(i % 3, i % 2)
        return 0
    lax.fori_loop(0, (nb + 1 + 5) // 6, stage_group, 0)
    plsc.subcore_barrier()

    base = s * DEG_SUB
    for k in range(DEG_SUB // DEG_CHUNK):
        off = base + jnp.minimum(k * DEG_CHUNK, zrows - DEG_CHUNK)
        pltpu.sync_copy(acc_sh.at[pl.ds(off, DEG_CHUNK)], dbuf)

        def ir(r, _):
            deg = dbuf[r, :]
            dbuf[r, :] = 1.0 / jnp.where(deg == 0.0, 1.0, deg)
            return 0
        lax.fori_loop(0, DEG_CHUNK, ir, 0)
        pltpu.sync_copy(dbuf, inv_ref.at[pl.ds(c * N + off, DEG_CHUNK)])


def _run_deg(ed):
    mesh = plsc.VectorSubcoreMesh(core_axis_name="c", subcore_axis_name="s")
    return pl.kernel(
        _deg_body,
        out_type=jax.ShapeDtypeStruct((2 * N, DEG_W), f32),
        mesh=mesh,
        scratch_types=[
            pltpu.VMEM_SHARED((N, DEG_W), f32),
            pltpu.VMEM((3, CH), i32),
            pltpu.VMEM((3, CH), i32),
            pltpu.VMEM((3, CH), i32),
            pltpu.VMEM((CH, DEG_W), f32),
            pltpu.VMEM((CH, DEG_W), f32),
            pltpu.VMEM((CH, DEG_W), f32),
            pltpu.VMEM((ZB, DEG_W), f32),
            pltpu.VMEM((DEG_CHUNK, DEG_W), f32),
            pltpu.SemaphoreType.DMA,
            pltpu.SemaphoreType.DMA,
            pltpu.SemaphoreType.DMA,
            pltpu.SemaphoreType.DMA,
            pltpu.SemaphoreType.DMA,
        ],
        compiler_params=pltpu.CompilerParams(use_tc_tiling_on_sc=False,
                                             needs_layout_passes=False),
    )(ed)


ACC_ROWS = 51200
OUT_SUB = 3200
OUT_SUB_LAST = HALF - 15 * OUT_SUB
ZB = 16
NB2 = E // CH


def _hop_body(x_ref, ed_ref, inv_ref, out_ref, acc_sh,
              e0, e1, e2, r0, r1, zbuf, inv_v,
              se0, se1, se2, sg0, sg1, ss0, ss1):
    c = lax.axis_index("c")
    s = lax.axis_index("s")
    iota = _iota16()
    zeros16f = jnp.zeros((16,), f32)
    coff = c * HALF
    ebufs, esems = [e0, e1, e2], [se0, se1, se2]
    rbufs, gsems, ssems = [r0, r1], [sg0, sg1], [ss0, ss1]

    def zb(i, _):
        zbuf[i, pl.ds(0, 16)] = zeros16f
        zbuf[i, pl.ds(16, 16)] = zeros16f
        return 0
    lax.fori_loop(0, ZB, zb, 0)
    zrows = ACC_ROWS // NS

    def zc(k, _):
        pltpu.sync_copy(zbuf, acc_sh.at[pl.ds(s * zrows + k * ZB, ZB)])
        return 0
    lax.fori_loop(0, zrows // ZB, zc, 0)
    plsc.subcore_barrier()

    nb = 781 + jnp.where(s < 4, 1, 0)
    start_s = s * 781 + jnp.minimum(s, 4)

    def fire_edge(b, ei):
        pltpu.async_copy(ed_ref.at[start_s + b], ebufs[ei], esems[ei])

    def wait_edge(ei):
        pltpu.make_async_copy(ed_ref.at[0], ebufs[ei], esems[ei]).wait()

    def fire_gather(ei, ri):
        pltpu.async_copy(x_ref.at[ebufs[ei].at[0]], rbufs[ri], gsems[ri])

    def wait_gather(ri):
        pltpu.make_async_copy(x_ref.at[pl.ds(0, CH)], rbufs[ri],
                              gsems[ri]).wait()

    def fire_scatter(ei, ri):
        pltpu.async_copy(rbufs[ri], acc_sh.at[ebufs[ei].at[1]], ssems[ri],
                         add=True)

    def wait_scatter(ri):
        pltpu.make_async_copy(rbufs[ri], acc_sh.at[pl.ds(0, CH)],
                              ssems[ri]).wait()

    def compute(ei, ri):
        es, rs = ebufs[ei], rbufs[ri]

        def wm(g, _):
            sl = pl.ds(g * 16, 16)
            dst = es[1, sl]
            local = dst - coff
            ok = (local >= 0) & (local < HALF)
            trash = HALF + s * 64 + ((g * 16 + iota) & 63)
            es[1, sl] = jnp.where(ok, local, trash)
            w16 = plsc.bitcast(es[2, sl], f32)
            for i in range(16):
                wsp = jnp.full((16,), w16[i], f32)
                r = g * 16 + i
                rs[r, pl.ds(0, 16)] = rs[r, pl.ds(0, 16)] * wsp
                rs[r, pl.ds(16, 16)] = rs[r, pl.ds(16, 16)] * wsp
            return 0
        lax.fori_loop(0, 8, wm, 0)

    fire_edge(0, 0)
    fire_edge(1, 1)
    wait_edge(0)
    fire_gather(0, 0)

    def stage_group(k, _):
        for i in range(6):
            b = k * 6 + i

            @pl.when(b + 1 < nb)
            def _():
                wait_edge((i + 1) % 3)

            @pl.when((b >= 1) & (b <= nb))
            def _():
                wait_scatter((i + 1) % 2)

            @pl.when(b + 2 < nb)
            def _():
                fire_edge(b + 2, (i + 2) % 3)

            @pl.when(b < nb)
            def _():
                wait_gather(i % 2)

            @pl.when(b + 1 < nb)
            def _():
                fire_gather((i + 1) % 3, (i + 1) % 2)

            @pl.when(b < nb)
            def _():
                compute(i % 3, i % 2)
                fire_scatter(i % 3, i % 2)
        return 0
    lax.fori_loop(0, (nb + 1 + 5) // 6, stage_group, 0)
    plsc.subcore_barrier()

    sub_base = s * OUT_SUB
    rows_s = jnp.where(s == 15, OUT_SUB_LAST, OUT_SUB)

    def out_chunk(k, _):
        cs = jnp.minimum(k * CH, rows_s - CH)
        pltpu.sync_copy(acc_sh.at[pl.ds(sub_base + cs, CH)], r0)
        pltpu.sync_copy(inv_ref.at[pl.ds(coff + sub_base + cs, CH)], inv_v)

        def sr(r, _):
            iv = inv_v[r, :]
            r0[r, pl.ds(0, 16)] = r0[r, pl.ds(0, 16)] * iv
            r0[r, pl.ds(16, 16)] = r0[r, pl.ds(16, 16)] * iv
            return 0
        lax.fori_loop(0, CH, sr, 0)
        pltpu.sync_copy(r0, out_ref.at[pl.ds(coff + sub_base + cs, CH)])
        return 0
    lax.fori_loop(0, OUT_SUB // CH, out_chunk, 0)


def _run_hop(x, ed, inv):
    mesh = plsc.VectorSubcoreMesh(core_axis_name="c", subcore_axis_name="s")
    return pl.kernel(
        _hop_body,
        out_type=jax.ShapeDtypeStruct((N, H), f32),
        mesh=mesh,
        scratch_types=[
            pltpu.VMEM_SHARED((ACC_ROWS, H), f32),
            pltpu.VMEM((3, CH), i32),
            pltpu.VMEM((3, CH), i32),
            pltpu.VMEM((3, CH), i32),
            pltpu.VMEM((CH, H), f32),
            pltpu.VMEM((CH, H), f32),
            pltpu.VMEM((ZB, H), f32),
            pltpu.VMEM((CH, DEG_W), f32),
            pltpu.SemaphoreType.DMA,
            pltpu.SemaphoreType.DMA,
            pltpu.SemaphoreType.DMA,
            pltpu.SemaphoreType.DMA,
            pltpu.SemaphoreType.DMA,
            pltpu.SemaphoreType.DMA,
            pltpu.SemaphoreType.DMA,
        ],
        compiler_params=pltpu.CompilerParams(use_tc_tiling_on_sc=False,
                                             needs_layout_passes=False),
    )(x, ed, inv)


def _dec_body(h_ref, f1_ref, f2_ref, b1_ref, b2_ref, wdiff_ref, wdec_ref,
              bdiff_ref, bdec_ref, out_ref):
    wdec = wdec_ref[...]
    wf = jnp.dot(wdiff_ref[...], wdec, preferred_element_type=f32)
    bias = jnp.dot(bdiff_ref[...], wdec, preferred_element_type=f32) \
        + bdec_ref[...]
    acc = bias
    parts = (h_ref, f1_ref, f2_ref, b1_ref, b2_ref)
    for p, ref in enumerate(parts):
        acc = acc + jnp.dot(ref[...], wf[p * H:(p + 1) * H, :],
                            preferred_element_type=f32)
    out_ref[...] = acc


def _run_dec(h, f1, f2, b1, b2, W_diff, W_dec, b_diff, b_dec):
    NB = 5000
    grid = N // NB
    part_spec = pl.BlockSpec((NB, H), lambda i: (i, 0))
    return pl.pallas_call(
        _dec_body,
        grid=(grid,),
        in_specs=[
            part_spec, part_spec, part_spec, part_spec, part_spec,
            pl.BlockSpec((5 * H, H), lambda i: (0, 0)),
            pl.BlockSpec((H, HORIZON), lambda i: (0, 0)),
            pl.BlockSpec((1, H), lambda i: (0, 0)),
            pl.BlockSpec((1, HORIZON), lambda i: (0, 0)),
        ],
        out_specs=pl.BlockSpec((NB, HORIZON), lambda i: (i, 0)),
        out_shape=jax.ShapeDtypeStruct((N, HORIZON), f32),
    )(h, f1, f2, b1, b2, W_diff, W_dec, b_diff, b_dec)


def kernel(x, edge_index, edge_weight, W_enc, b_enc, node_emb, Wi, Wh, bi,
           bh, W_diff, b_diff, W_dec, b_dec):
    x2 = jnp.transpose(x.reshape(T, N))
    src2 = edge_index[0].reshape(NB2, 1, CH)
    dst2 = edge_index[1].reshape(NB2, 1, CH)
    wb = lax.bitcast_convert_type(edge_weight, i32).reshape(NB2, 1, CH)
    ed_f = jnp.concatenate([src2, dst2, wb], axis=1)
    ed_b = jnp.concatenate([dst2, src2, wb], axis=1)

    h = _run_gru(x2, node_emb, W_enc, b_enc.reshape(1, H), Wi, Wh,
                 bi.reshape(1, 3 * H), bh.reshape(1, 3 * H))

    inv2 = _run_deg(ed_f)
    inv_f = inv2[:N]
    inv_b = inv2[N:]

    f1 = _run_hop(h, ed_f, inv_f)
    f2 = _run_hop(f1, ed_f, inv_f)
    b1 = _run_hop(h, ed_b, inv_b)
    b2 = _run_hop(b1, ed_b, inv_b)

    outn = _run_dec(h, f1, f2, b1, b2, W_diff, W_dec,
                    b_diff.reshape(1, H), b_dec.reshape(1, HORIZON))
    return jnp.transpose(outn.reshape(1, N, HORIZON, 1), (0, 2, 1, 3))
